# Initial kernel scaffold; baseline (speedup 1.0000x reference)
import functools

import jax
import jax.numpy as jnp
from jax import lax
from jax.experimental import pallas as pl
from jax.experimental.pallas import tpu as pltpu
from jax.experimental.pallas import tpu_sc as plsc

_NC = 2
_NS = 16
_NW = _NC * _NS

_BLK = 1024


def _mm(a, w):
    return lax.dot_general(a, w, (((1,), (0,)), ((), ())),
                           preferred_element_type=jnp.float32)


def _ln(h, g, b):
    m = jnp.mean(h, axis=-1, keepdims=True)
    c = h - m
    v = jnp.mean(c * c, axis=-1, keepdims=True)
    return c * lax.rsqrt(v + 1e-5) * g + b


def _relu(h):
    return jnp.maximum(h, 0.0)


def _init_body(x_ref, w1, b1, w2, b2, w3, b3, lg, lb, o_ref):
    h = _relu(_mm(x_ref[...], w1[...]) + b1[...])
    h = _relu(_mm(h, w2[...]) + b2[...])
    h = _mm(h, w3[...]) + b3[...]
    o_ref[...] = _ln(h, lg[...], lb[...])


def _iter_body(n_rows, x_ref, ub_ref,
               nw1x, nw1u, nb1, nw2, nb2, nw3, nb3,
               ng0g, ng0b, ng1g, ng1b, ng2g, ng2b,
               pw1x, pw1u, pb1, pw2, pb2, pw3, pb3, png, pnb,
               xo_ref, g_ref):
    x = x_ref[...]
    ub = ub_ref[...]
    h = _mm(x, nw1x[...]) + _mm(ub, nw1u[...]) + nb1[...]
    h = _relu(_ln(h, ng0g[...], ng0b[...]))
    h = _relu(_ln(_mm(h, nw2[...]) + nb2[...], ng1g[...], ng1b[...]))
    h = _ln(_mm(h, nw3[...]) + nb3[...], ng2g[...], ng2b[...])
    xn = h + x
    xo_ref[...] = xn
    g = _relu(_mm(xn, pw1x[...]) + _mm(ub, pw1u[...]) + pb1[...])
    g = _relu(_mm(g, pw2[...]) + pb2[...])
    g = _ln(_mm(g, pw3[...]) + pb3[...], png[...], pnb[...])
    i = pl.program_id(0)
    rows = i * _BLK + lax.broadcasted_iota(jnp.int32, g.shape, 0)
    g_ref[...] = jnp.where(rows < n_rows, g, 0.0)


def _post_body(parts_ref, u_ref, w1a, w1u, b1, w2, b2, w3, b3, lg, lb, o_ref):
    agg = parts_ref[0] + parts_ref[1]
    u = u_ref[...]
    q = _relu(_mm(agg, w1a[...]) + _mm(u, w1u[...]) + b1[...])
    q = _relu(_mm(q, w2[...]) + b2[...])
    q = _ln(_mm(q, w3[...]) + b3[...], lg[...], lb[...])
    o_ref[...] = q + u


def _head_body(x_ref, w1, b1, w2, b2, w3, b3, o_ref):
    h = _relu(_mm(x_ref[...], w1[...]) + b1[...])
    h = _relu(_mm(h, w2[...]) + b2[...])
    o_ref[...] = _mm(h, w3[...]) + b3[...]


def _full(a):
    nd = a.ndim
    return pl.BlockSpec(a.shape, lambda i, _n=nd: (0,) * _n)


def _row2(v):
    return v.reshape(1, -1)


def _make_gather(n_pad, d, chunk):
    rpw = n_pad // _NW
    n_ch = rpw // chunk
    mesh = plsc.VectorSubcoreMesh(core_axis_name="c", subcore_axis_name="s")

    @functools.partial(
        pl.kernel, mesh=mesh,
        out_type=jax.ShapeDtypeStruct((n_pad, d), jnp.float32),
        scratch_types=[
            pltpu.VMEM((chunk,), jnp.int32),
            pltpu.VMEM((chunk, d), jnp.float32),
            pltpu.SemaphoreType.DMA,
        ],
    )
    def gather_k(u_hbm, idx_hbm, out_hbm, idx_v, rows_v, sem):
        wid = lax.axis_index("s") * _NC + lax.axis_index("c")
        base = wid * rpw
        for c in range(n_ch):
            off = base + c * chunk
            pltpu.sync_copy(idx_hbm.at[pl.ds(off, chunk)], idx_v)
            pltpu.async_copy(u_hbm.at[idx_v], rows_v, sem).wait()
            pltpu.sync_copy(rows_v, out_hbm.at[pl.ds(off, chunk)])

    return gather_k


def _make_scatter(n_pad, b, d, chunk):
    rpw = n_pad // _NW
    n_ch = rpw // chunk
    mesh = plsc.VectorSubcoreMesh(core_axis_name="c", subcore_axis_name="s")

    @functools.partial(
        pl.kernel, mesh=mesh,
        out_type=jax.ShapeDtypeStruct((_NC, b, d), jnp.float32),
        scratch_types=[
            pltpu.VMEM((chunk,), jnp.int32),
            pltpu.VMEM((chunk, d), jnp.float32),
            pltpu.VMEM_SHARED((b, d), jnp.float32),
            pltpu.SemaphoreType.DMA,
        ],
    )
    def scatter_k(g_hbm, idx_hbm, zeros_hbm, out_hbm, idx_v, rows_v, acc, sem):
        cid = lax.axis_index("c")
        sid = lax.axis_index("s")
        wid = sid * _NC + cid

        @pl.when(sid == 0)
        def _():
            pltpu.sync_copy(zeros_hbm, acc)

        plsc.subcore_barrier()
        for c in range(n_ch):
            off = wid * rpw + c * chunk
            pltpu.sync_copy(idx_hbm.at[pl.ds(off, chunk)], idx_v)
            pltpu.sync_copy(g_hbm.at[pl.ds(off, chunk)], rows_v)
            pltpu.sync_copy(rows_v, acc.at[idx_v], add=True)
        plsc.subcore_barrier()

        @pl.when(sid == 0)
        def _():
            pltpu.sync_copy(acc, out_hbm.at[cid])

    return scatter_k


def kernel(x, batch, y, params):
    n, d_in = x.shape
    b = y.shape[0]
    d_loc = params["init"]["l"][2]["w"].shape[1]
    d_glo = params["layers"][0]["post"]["l"][2]["w"].shape[1]

    n_pad = -(-n // _BLK) * _BLK
    grid = n_pad // _BLK
    chunk = (n_pad // _NW) // 2

    batch_p = jnp.pad(batch, (0, n_pad - n))
    zeros_bd = jnp.zeros((b, d_glo), jnp.float32)

    gather_fn = _make_gather(n_pad, d_glo, chunk)
    scatter_fn = _make_scatter(n_pad, b, d_glo, chunk)

    ip = params["init"]
    init_args = [ip["l"][0]["w"], _row2(ip["l"][0]["b"]),
                 ip["l"][1]["w"], _row2(ip["l"][1]["b"]),
                 ip["l"][2]["w"], _row2(ip["l"][2]["b"]),
                 _row2(ip["n"]["g"]), _row2(ip["n"]["b"])]
    x0 = pl.pallas_call(
        _init_body,
        grid=(grid,),
        in_specs=[pl.BlockSpec((_BLK, d_in), lambda i: (i, 0))]
                 + [_full(a) for a in init_args],
        out_specs=pl.BlockSpec((_BLK, d_loc), lambda i: (i, 0)),
        out_shape=jax.ShapeDtypeStruct((n_pad, d_loc), jnp.float32),
    )(x, *init_args)

    xs = x0
    u = jnp.zeros((b, d_glo), jnp.float32)

    for lp in params["layers"]:
        ub = gather_fn(u, batch_p)

        npar, pp = lp["node"], lp["pre"]
        iter_args = [
            npar["l"][0]["w"][:d_loc], npar["l"][0]["w"][d_loc:],
            _row2(npar["l"][0]["b"]),
            npar["l"][1]["w"], _row2(npar["l"][1]["b"]),
            npar["l"][2]["w"], _row2(npar["l"][2]["b"]),
            _row2(npar["n"][0]["g"]), _row2(npar["n"][0]["b"]),
            _row2(npar["n"][1]["g"]), _row2(npar["n"][1]["b"]),
            _row2(npar["n"][2]["g"]), _row2(npar["n"][2]["b"]),
            pp["l"][0]["w"][:d_loc], pp["l"][0]["w"][d_loc:],
            _row2(pp["l"][0]["b"]),
            pp["l"][1]["w"], _row2(pp["l"][1]["b"]),
            pp["l"][2]["w"], _row2(pp["l"][2]["b"]),
            _row2(pp["n"]["g"]), _row2(pp["n"]["b"]),
        ]
        xs, g = pl.pallas_call(
            functools.partial(_iter_body, n),
            grid=(grid,),
            in_specs=[pl.BlockSpec((_BLK, d_loc), lambda i: (i, 0)),
                      pl.BlockSpec((_BLK, d_glo), lambda i: (i, 0))]
                     + [_full(a) for a in iter_args],
            out_specs=[pl.BlockSpec((_BLK, d_loc), lambda i: (i, 0)),
                       pl.BlockSpec((_BLK, d_glo), lambda i: (i, 0))],
            out_shape=[jax.ShapeDtypeStruct((n_pad, d_loc), jnp.float32),
                       jax.ShapeDtypeStruct((n_pad, d_glo), jnp.float32)],
        )(xs, ub, *iter_args)

        parts = scatter_fn(g, batch_p, zeros_bd)

        qp = lp["post"]
        post_args = [
            qp["l"][0]["w"][:d_glo], qp["l"][0]["w"][d_glo:],
            _row2(qp["l"][0]["b"]),
            qp["l"][1]["w"], _row2(qp["l"][1]["b"]),
            qp["l"][2]["w"], _row2(qp["l"][2]["b"]),
            _row2(qp["n"]["g"]), _row2(qp["n"]["b"]),
        ]
        u = pl.pallas_call(
            _post_body,
            in_specs=[pl.BlockSpec((_NC, b, d_glo), lambda: (0, 0, 0)),
                      pl.BlockSpec((b, d_glo), lambda: (0, 0))]
                     + [pl.BlockSpec(a.shape, lambda _a=a: (0,) * _a.ndim)
                        for a in post_args],
            out_specs=pl.BlockSpec((b, d_glo), lambda: (0, 0)),
            out_shape=jax.ShapeDtypeStruct((b, d_glo), jnp.float32),
        )(parts, u, *post_args)

    ol = params["out_local"]
    d_out_l = ol[2]["w"].shape[1]
    head_args = [ol[0]["w"], _row2(ol[0]["b"]),
                 ol[1]["w"], _row2(ol[1]["b"]),
                 ol[2]["w"], _row2(ol[2]["b"])]
    xo = pl.pallas_call(
        _head_body,
        grid=(grid,),
        in_specs=[pl.BlockSpec((_BLK, d_loc), lambda i: (i, 0))]
                 + [_full(a) for a in head_args],
        out_specs=pl.BlockSpec((_BLK, d_out_l), lambda i: (i, 0)),
        out_shape=jax.ShapeDtypeStruct((n, d_out_l), jnp.float32),
    )(xs, *head_args)

    og = params["out_global"]
    d_out_g = og[2]["w"].shape[1]
    g_args = [og[0]["w"], _row2(og[0]["b"]),
              og[1]["w"], _row2(og[1]["b"]),
              og[2]["w"], _row2(og[2]["b"])]
    uo = pl.pallas_call(
        _head_body,
        in_specs=[pl.BlockSpec((b, d_glo), lambda: (0, 0))]
                 + [pl.BlockSpec(a.shape, lambda _a=a: (0,) * _a.ndim)
                    for a in g_args],
        out_specs=pl.BlockSpec((b, d_out_g), lambda: (0, 0)),
        out_shape=jax.ShapeDtypeStruct((b, d_out_g), jnp.float32),
    )(u, *g_args)

    return (xo, uo)


# speedup vs baseline: 2.4813x; 2.4813x over previous
import functools

import jax
import jax.numpy as jnp
from jax import lax
from jax.experimental import pallas as pl
from jax.experimental.pallas import tpu as pltpu
from jax.experimental.pallas import tpu_sc as plsc

_NC = 2
_NS = 16
_NW = _NC * _NS

_BLK = 1024
_W = 128


def _mm(a, w):
    return lax.dot_general(a, w, (((1,), (0,)), ((), ())),
                           preferred_element_type=jnp.float32)


def _ln2(h, g, b, j2):
    mb = _mm(h, j2)
    vb = _mm(h * h, j2)
    return (h - mb) * lax.rsqrt(vb - mb * mb + 1e-5) * g + b


def _relu(h):
    return jnp.maximum(h, 0.0)


def _combine(ue, uor, d):
    lane = lax.broadcasted_iota(jnp.int32, ue.shape, 1)
    left = lane < d
    un = jnp.where(left, ue, uor)
    up = jnp.roll(jnp.where(left, uor, ue), d, axis=1)
    return un, up


def _iter_common(x, un, up, refs):
    (nb1, nw2, nb2, nw3, nb3,
     ng0g, ng0b, ng1g, ng1b, ng2g, ng2b,
     nw1x, pw1x, pb1, pw2, pb2, pw3, pb3, png, pnb, j2) = refs
    jv = j2[...]
    h = _mm(x, nw1x[...]) + un + nb1[...]
    h = _relu(_ln2(h, ng0g[...], ng0b[...], jv))
    h = _relu(_ln2(_mm(h, nw2[...]) + nb2[...], ng1g[...], ng1b[...], jv))
    h = _ln2(_mm(h, nw3[...]) + nb3[...], ng2g[...], ng2b[...], jv)
    xn = h + x
    g = _relu(_mm(xn, pw1x[...]) + up + pb1[...])
    g = _relu(_mm(g, pw2[...]) + pb2[...])
    g = _ln2(_mm(g, pw3[...]) + pb3[...], png[...], pnb[...], jv)
    return xn, g


def _emit(n2_rows, xn, g, xo_ref, ga_ref):
    xo_ref[...] = xn
    rows = lax.broadcasted_iota(jnp.int32, (g.shape[0], 1), 0)
    ga_ref[...] = jnp.where(pl.program_id(0) * _BLK + rows < n2_rows, g, 0.0)


def _iter_body_first(n2_rows, d_loc, x_ref, iw1, ib1, iw2, ib2, iw3, ib3,
                     ilg, ilb, *refs):
    xo_ref, ga_ref = refs[-2:]
    h = _relu(_mm(x_ref[...], iw1[...]) + ib1[...])
    h = _relu(_mm(h, iw2[...]) + ib2[...])
    h = _mm(h, iw3[...]) + ib3[...]
    x0 = _ln2(h, ilg[...], ilb[...], refs[-3][...])
    xn, g = _iter_common(x0, 0.0, 0.0, refs[:-2])
    _emit(n2_rows, xn, g, xo_ref, ga_ref)


def _iter_body(n2_rows, d_loc, xg_ref, ue_ref, uo_ref, *refs):
    xo_ref, ga_ref = refs[-2:]
    un, up = _combine(ue_ref[...], uo_ref[...], d_loc)
    xn, g = _iter_common(xg_ref[...], un, up, refs[:-2])
    _emit(n2_rows, xn, g, xo_ref, ga_ref)


def _iter_body_last(n2_rows, d_loc, xg_ref, ue_ref, uo_ref, *refs):
    (hw1, hb1, hw2, hb2, hw3, hb3, ga_ref, xo_ref) = refs[-8:]
    un, up = _combine(ue_ref[...], uo_ref[...], d_loc)
    xn, g = _iter_common(xg_ref[...], un, up, refs[:-8])
    rows = lax.broadcasted_iota(jnp.int32, (g.shape[0], 1), 0)
    ga_ref[...] = jnp.where(pl.program_id(0) * _BLK + rows < n2_rows, g, 0.0)
    h = _relu(_mm(xn, hw1[...]) + hb1[...])
    h = _relu(_mm(h, hw2[...]) + hb2[...])
    xo_ref[...] = _mm(h, hw3[...]) + hb3[...]


def _post_body(d_glo, parts_ref, u_ref,
               w1a, w1u, b1, w2, b2, w3, b3, lg, lb,
               wnn, wpn, j2, ou_ref, op_ref, opr_ref):
    agg = (parts_ref[0, 0, :, :d_glo] + parts_ref[1, 0, :, :d_glo]
           + parts_ref[0, 1, :, d_glo:] + parts_ref[1, 1, :, d_glo:])
    u = u_ref[...]
    q = _relu(_mm(agg, w1a[...]) + _mm(u, w1u[...]) + b1[...])
    q = _relu(_mm(q, w2[...]) + b2[...])
    q = _ln2(_mm(q, w3[...]) + b3[...], lg[...], lb[...], j2[...])
    un = q + u
    ou_ref[...] = un
    pn = _mm(un, wnn[...])
    pp = _mm(un, wpn[...])
    op_ref[...] = jnp.concatenate([pn, pp], axis=-1)
    opr_ref[...] = jnp.concatenate([pp, pn], axis=-1)


def _head_body(x_ref, w1, b1, w2, b2, w3, b3, o_ref):
    h = _relu(_mm(x_ref[...], w1[...]) + b1[...])
    h = _relu(_mm(h, w2[...]) + b2[...])
    o_ref[...] = _mm(h, w3[...]) + b3[...]


def _full(a):
    nd = a.ndim
    return pl.BlockSpec(a.shape, lambda i, _n=nd: (0,) * _n)


def _row2(v):
    return v.reshape(1, -1)


def _k2(w):
    z = jnp.zeros_like(w)
    return jnp.concatenate(
        [jnp.concatenate([w, z], axis=1), jnp.concatenate([z, w], axis=1)],
        axis=0)


def _t2(v):
    return jnp.concatenate([v, v]).reshape(1, -1)


def _make_gather(n2_pad, chunk):
    rpw = n2_pad // _NW
    n_ch = rpw // chunk
    mesh = plsc.VectorSubcoreMesh(core_axis_name="c", subcore_axis_name="s")

    @functools.partial(
        pl.kernel, mesh=mesh,
        out_type=[jax.ShapeDtypeStruct((n2_pad, _W), jnp.float32),
                  jax.ShapeDtypeStruct((n2_pad, _W), jnp.float32)],
        scratch_types=[
            pltpu.VMEM((chunk,), jnp.int32),
            pltpu.VMEM((chunk, _W), jnp.float32),
            pltpu.SemaphoreType.DMA,
        ],
    )
    def gather_k(p_hbm, pr_hbm, idx_e_hbm, idx_o_hbm, oute_hbm, outo_hbm,
                 idx_v, rows_v, sem):
        wid = lax.axis_index("s") * _NC + lax.axis_index("c")
        base = wid * rpw
        for tab_hbm, idx_hbm, out_hbm in ((p_hbm, idx_e_hbm, oute_hbm),
                                          (pr_hbm, idx_o_hbm, outo_hbm)):
            for c in range(n_ch):
                off = base + c * chunk
                pltpu.sync_copy(idx_hbm.at[pl.ds(off, chunk)], idx_v)
                pltpu.async_copy(tab_hbm.at[idx_v], rows_v, sem).wait()
                pltpu.sync_copy(rows_v, out_hbm.at[pl.ds(off, chunk)])

    return gather_k


def _make_scatter(n2_pad, b, chunk):
    rpw = n2_pad // _NW
    n_ch = rpw // chunk
    mesh = plsc.VectorSubcoreMesh(core_axis_name="c", subcore_axis_name="s")

    @functools.partial(
        pl.kernel, mesh=mesh,
        out_type=jax.ShapeDtypeStruct((_NC, 2, b, _W), jnp.float32),
        scratch_types=[
            pltpu.VMEM((chunk,), jnp.int32),
            pltpu.VMEM((chunk,), jnp.int32),
            pltpu.VMEM((chunk, _W), jnp.float32),
            pltpu.VMEM_SHARED((2, b, _W), jnp.float32),
            pltpu.SemaphoreType.DMA,
        ],
    )
    def scatter_k(ga_hbm, idx_e_hbm, idx_o_hbm, zeros_hbm, out_hbm,
                  idxe_v, idxo_v, rows_v, acc, sem):
        cid = lax.axis_index("c")
        sid = lax.axis_index("s")
        wid = sid * _NC + cid

        @pl.when(sid == 0)
        def _():
            pltpu.sync_copy(zeros_hbm, acc)

        plsc.subcore_barrier()
        for c in range(n_ch):
            off = wid * rpw + c * chunk
            pltpu.sync_copy(idx_e_hbm.at[pl.ds(off, chunk)], idxe_v)
            pltpu.sync_copy(idx_o_hbm.at[pl.ds(off, chunk)], idxo_v)
            pltpu.sync_copy(ga_hbm.at[pl.ds(off, chunk)], rows_v)
            pltpu.sync_copy(rows_v, acc.at[0].at[idxe_v], add=True)
            pltpu.sync_copy(rows_v, acc.at[1].at[idxo_v], add=True)
        plsc.subcore_barrier()

        @pl.when(sid == 0)
        def _():
            pltpu.sync_copy(acc, out_hbm.at[cid])

    return scatter_k


def kernel(x, batch, y, params):
    n, d_in = x.shape
    b = y.shape[0]
    d_loc = params["init"]["l"][2]["w"].shape[1]
    d_glo = params["layers"][0]["post"]["l"][2]["w"].shape[1]

    n2 = n // 2
    n2_pad = -(-n2 // _BLK) * _BLK
    n_pad = 2 * n2_pad
    grid = n2_pad // _BLK
    chunk = (n2_pad // _NW) // 2

    batch_p = jnp.pad(batch, (0, n_pad - n))
    batch_e = batch_p[0::2]
    batch_o = batch_p[1::2]
    zeros_bw = jnp.zeros((2, b, _W), jnp.float32)

    gather_fn = _make_gather(n2_pad, chunk)
    scatter_fn = _make_scatter(n2_pad, b, chunk)

    j2c = jnp.kron(jnp.eye(2, dtype=jnp.float32),
                   jnp.full((d_loc, d_loc), 1.0 / d_loc, jnp.float32))

    def iter_args_of(lp):
        npar, pp = lp["node"], lp["pre"]
        return [
            _t2(npar["l"][0]["b"]),
            _k2(npar["l"][1]["w"]), _t2(npar["l"][1]["b"]),
            _k2(npar["l"][2]["w"]), _t2(npar["l"][2]["b"]),
            _t2(npar["n"][0]["g"]), _t2(npar["n"][0]["b"]),
            _t2(npar["n"][1]["g"]), _t2(npar["n"][1]["b"]),
            _t2(npar["n"][2]["g"]), _t2(npar["n"][2]["b"]),
            _k2(npar["l"][0]["w"][:d_loc]),
            _k2(pp["l"][0]["w"][:d_loc]), _t2(pp["l"][0]["b"]),
            _k2(pp["l"][1]["w"]), _t2(pp["l"][1]["b"]),
            _k2(pp["l"][2]["w"]), _t2(pp["l"][2]["b"]),
            _t2(pp["n"]["g"]), _t2(pp["n"]["b"]),
            j2c,
        ]

    ip = params["init"]
    init_args = [_k2(ip["l"][0]["w"]), _t2(ip["l"][0]["b"]),
                 _k2(ip["l"][1]["w"]), _t2(ip["l"][1]["b"]),
                 _k2(ip["l"][2]["w"]), _t2(ip["l"][2]["b"]),
                 _t2(ip["n"]["g"]), _t2(ip["n"]["b"])]
    ol = params["out_local"]
    d_out_l = ol[2]["w"].shape[1]
    head_args = [_k2(ol[0]["w"]), _t2(ol[0]["b"]),
                 _k2(ol[1]["w"]), _t2(ol[1]["b"]),
                 _k2(ol[2]["w"]), _t2(ol[2]["b"])]
    x_f = x.reshape(n2, 2 * d_in)

    u = jnp.zeros((b, d_glo), jnp.float32)
    proj = None
    projr = None
    x2 = None
    xo2 = None

    layers = params["layers"]
    n_spec = pl.BlockSpec((_BLK, _W), lambda i: (i, 0))
    for li, lp in enumerate(layers):
        iter_args = iter_args_of(lp)
        last = li == len(layers) - 1
        if li == 0:
            body = functools.partial(_iter_body_first, n2, d_loc)
            data = [x_f, *init_args]
            data_specs = ([pl.BlockSpec((_BLK, 2 * d_in), lambda i: (i, 0))]
                          + [_full(a) for a in init_args])
        else:
            ue, uo = gather_fn(proj, projr, batch_e, batch_o)
            body = functools.partial(
                _iter_body_last if last else _iter_body, n2, d_loc)
            data = [x2, ue, uo]
            data_specs = [n_spec] * 3
        if last:
            ga, xo2 = pl.pallas_call(
                body,
                grid=(grid,),
                in_specs=(data_specs + [_full(a) for a in iter_args]
                          + [_full(a) for a in head_args]),
                out_specs=[n_spec,
                           pl.BlockSpec((_BLK, 2 * d_out_l),
                                        lambda i: (i, 0))],
                out_shape=[jax.ShapeDtypeStruct((n2_pad, _W), jnp.float32),
                           jax.ShapeDtypeStruct((n2_pad, 2 * d_out_l),
                                                jnp.float32)],
            )(*data, *iter_args, *head_args)
        else:
            x2, ga = pl.pallas_call(
                body,
                grid=(grid,),
                in_specs=data_specs + [_full(a) for a in iter_args],
                out_specs=[n_spec] * 2,
                out_shape=[jax.ShapeDtypeStruct((n2_pad, _W),
                                                jnp.float32)] * 2,
            )(*data, *iter_args)

        parts = scatter_fn(ga, batch_e, batch_o, zeros_bw)

        nxt = layers[(li + 1) % len(layers)]
        qp = lp["post"]
        post_args = [
            qp["l"][0]["w"][:d_glo], qp["l"][0]["w"][d_glo:],
            _row2(qp["l"][0]["b"]),
            qp["l"][1]["w"], _row2(qp["l"][1]["b"]),
            qp["l"][2]["w"], _row2(qp["l"][2]["b"]),
            _row2(qp["n"]["g"]), _row2(qp["n"]["b"]),
            nxt["node"]["l"][0]["w"][d_loc:],
            nxt["pre"]["l"][0]["w"][d_loc:],
            jnp.full((d_glo, d_glo), 1.0 / d_glo, jnp.float32),
        ]
        u, proj, projr = pl.pallas_call(
            functools.partial(_post_body, d_glo),
            in_specs=[pl.BlockSpec((_NC, 2, b, _W), lambda: (0, 0, 0, 0)),
                      pl.BlockSpec((b, d_glo), lambda: (0, 0))]
                     + [pl.BlockSpec(a.shape, lambda _a=a: (0,) * _a.ndim)
                        for a in post_args],
            out_specs=[pl.BlockSpec((b, d_glo), lambda: (0, 0)),
                       pl.BlockSpec((b, _W), lambda: (0, 0)),
                       pl.BlockSpec((b, _W), lambda: (0, 0))],
            out_shape=[jax.ShapeDtypeStruct((b, d_glo), jnp.float32),
                       jax.ShapeDtypeStruct((b, _W), jnp.float32),
                       jax.ShapeDtypeStruct((b, _W), jnp.float32)],
        )(parts, u, *post_args)

    xo = xo2.reshape(n_pad, d_out_l)[:n]

    og = params["out_global"]
    d_out_g = og[2]["w"].shape[1]
    g_args = [og[0]["w"], _row2(og[0]["b"]),
              og[1]["w"], _row2(og[1]["b"]),
              og[2]["w"], _row2(og[2]["b"])]
    uo = pl.pallas_call(
        _head_body,
        in_specs=[pl.BlockSpec((b, d_glo), lambda: (0, 0))]
                 + [pl.BlockSpec(a.shape, lambda _a=a: (0,) * _a.ndim)
                    for a in g_args],
        out_specs=pl.BlockSpec((b, d_out_g), lambda: (0, 0)),
        out_shape=jax.ShapeDtypeStruct((b, d_out_g), jnp.float32),
    )(u, *g_args)

    return (xo, uo)
